# baseline (speedup 1.0000x reference)
import functools

import jax
import jax.numpy as jnp
from jax import lax
from jax.experimental import pallas as pl
from jax.experimental.pallas import tpu as pltpu
from jax.experimental.pallas import tpu_sc as plsc

_B = 16
_Q = 20000
_NW = 32
_CH = 2560
_NV = _CH // 16
_F = _CH + 128
_QT = 19968
_SCALE = 1.0 / 512.0

_mesh = plsc.VectorSubcoreMesh(core_axis_name="c", subcore_axis_name="s")


@functools.partial(
    pl.kernel,
    mesh=_mesh,
    out_type=jax.ShapeDtypeStruct((5, _B * _Q), jnp.float32),
    scratch_types=[
        pltpu.VMEM((8, _F), jnp.float32),
        pltpu.VMEM((8, _CH), jnp.float32),
        pltpu.VMEM((8, 32), jnp.float32),
        pltpu.VMEM((5, _CH), jnp.float32),
    ],
    compiler_params=pltpu.CompilerParams(use_tc_tiling_on_sc=True),
)
def _sc_body(vt_hbm, tail_hbm, out_hbm, buf1, buf2, buft, buf_out):
    wid = lax.axis_index("s") * 2 + lax.axis_index("c")
    for r in range(4):
        c0 = (r * _NW + wid) * _CH

        @pl.when(c0 < _B * _Q)
        def _round():
            b0 = c0 // _Q
            q0 = c0 - b0 * _Q
            b1 = jnp.minimum(b0 + 1, _B - 1)
            i1 = jnp.clip(_QT - q0, 0, _CH) // 16
            i2 = jnp.clip(_Q - q0, 0, _CH) // 16
            qa = jnp.minimum((q0 // 128) * 128, _QT - _F)
            s0 = q0 - qa

            @pl.when(i1 > 0)
            def _():
                pltpu.sync_copy(vt_hbm.at[b0, :, pl.ds(qa, _F)], buf1)

            @pl.when(i1 < i2)
            def _():
                pltpu.sync_copy(tail_hbm.at[b0], buft)

            @pl.when(i2 < _NV)
            def _():
                pltpu.sync_copy(vt_hbm.at[b1, :, pl.ds(0, _CH)], buf2)

            bf0 = b0.astype(jnp.float32)
            bf1 = b1.astype(jnp.float32)

            def _compute(buf, i_to_off, bf, lo, hi):
                def _vec(i, carry):
                    sl = pl.ds(i_to_off(i), 16)
                    x1 = buf[pl.ds(1, 1), sl] * _SCALE
                    y1 = buf[pl.ds(2, 1), sl] * _SCALE
                    x2 = buf[pl.ds(3, 1), sl] * _SCALE
                    y2 = buf[pl.ds(4, 1), sl] * _SCALE
                    osl = pl.ds(i * 16, 16)
                    buf_out[pl.ds(0, 1), osl] = jnp.zeros((1, 16), jnp.float32) + bf
                    buf_out[pl.ds(1, 1), osl] = jnp.clip(jnp.minimum(x1, x2), 0.0, 1.0)
                    buf_out[pl.ds(2, 1), osl] = jnp.clip(jnp.minimum(y1, y2), 0.0, 1.0)
                    buf_out[pl.ds(3, 1), osl] = jnp.clip(jnp.maximum(x1, x2), 0.0, 1.0)
                    buf_out[pl.ds(4, 1), osl] = jnp.clip(jnp.maximum(y1, y2), 0.0, 1.0)
                    return carry

                lax.fori_loop(lo, hi, _vec, 0)

            _compute(buf1, lambda i: s0 + i * 16, bf0, 0, i1)
            _compute(buft, lambda i: (i - i1) * 16, bf0, i1, i2)
            _compute(buf2, lambda i: (i - i2) * 16, bf1, i2, _NV)

            pltpu.sync_copy(buf_out, out_hbm.at[:, pl.ds(c0, _CH)])


def kernel(vit_output, input_images_or_features):
    del input_images_or_features
    vt = jnp.transpose(vit_output, (0, 2, 1))
    tail = lax.slice(vt, (0, 0, _QT), (_B, 8, _Q))
    out = _sc_body(vt, tail)
    return out.T


# speedup vs baseline: 1.2632x; 1.2632x over previous
import functools

import jax
import jax.numpy as jnp
from jax import lax
from jax.experimental import pallas as pl
from jax.experimental.pallas import tpu as pltpu
from jax.experimental.pallas import tpu_sc as plsc

_B = 16
_Q = 20000
_N = _B * _Q
_NW = 32
_CH = 2560
_NV = _CH // 16
_F = _CH + 128
_QT = 19968
_NR = 4
_SCALE = 1.0 / 512.0

_mesh = plsc.VectorSubcoreMesh(core_axis_name="c", subcore_axis_name="s")


@functools.partial(
    pl.kernel,
    mesh=_mesh,
    out_type=jax.ShapeDtypeStruct((5, _N), jnp.float32),
    scratch_types=[
        pltpu.VMEM((2, 8, _F), jnp.float32),
        pltpu.VMEM((2, 8, _CH), jnp.float32),
        pltpu.VMEM((2, 8, 32), jnp.float32),
        pltpu.VMEM((2, 5, _CH), jnp.float32),
        pltpu.SemaphoreType.DMA((2,)),
        pltpu.SemaphoreType.DMA((2,)),
        pltpu.SemaphoreType.DMA((2,)),
        pltpu.SemaphoreType.DMA((2,)),
    ],
    compiler_params=pltpu.CompilerParams(use_tc_tiling_on_sc=True),
)
def _sc_body(vt_hbm, tail_hbm, out_hbm, bin1, bin2, bint, bout,
             sem1, sem2, semt, semo):
    wid = lax.axis_index("s") * 2 + lax.axis_index("c")

    def params(r):
        c0 = (r * _NW + wid) * _CH
        act = c0 < _N
        b0 = jnp.minimum(c0 // _Q, _B - 1)
        q0 = c0 - b0 * _Q
        b1 = jnp.minimum(b0 + 1, _B - 1)
        i1 = jnp.clip(_QT - q0, 0, _CH) // 16
        i2 = jnp.clip(_Q - q0, 0, _CH) // 16
        qa = jnp.minimum((q0 // 128) * 128, _QT - _F)
        return c0, act, b0, q0, b1, i1, i2, qa

    def in_copies(r, w):
        c0, act, b0, q0, b1, i1, i2, qa = params(r)
        yield act & (i1 > 0), pltpu.make_async_copy(
            vt_hbm.at[b0, :, pl.ds(qa, _F)], bin1.at[w], sem1.at[w])
        yield act & (i1 < i2), pltpu.make_async_copy(
            tail_hbm.at[b0], bint.at[w], semt.at[w])
        yield act & (i2 < _NV), pltpu.make_async_copy(
            vt_hbm.at[b1, :, pl.ds(0, _CH)], bin2.at[w], sem2.at[w])

    def fetch(r, w):
        for cond, copy in in_copies(r, w):
            pl.when(cond)(copy.start)

    def wait_in(r, w):
        for cond, copy in in_copies(r, w):
            pl.when(cond)(copy.wait)

    def out_copy(r, w):
        c0 = (r * _NW + wid) * _CH
        return pltpu.make_async_copy(
            bout.at[w], out_hbm.at[:, pl.ds(c0, _CH)], semo.at[w])

    fetch(0, 0)
    fetch(1, 1)
    for r in range(_NR):
        w = r % 2
        c0, act, b0, q0, b1, i1, i2, qa = params(r)
        if r >= 2:
            _, pact, *_ = params(r - 2)
            pl.when(pact)(out_copy(r - 2, w).wait)

        @pl.when(act)
        def _round():
            wait_in(r, w)
            s0 = q0 - qa
            bf0 = b0.astype(jnp.float32)
            bf1 = b1.astype(jnp.float32)
            ob = bout.at[w]

            def vec_body(buf, off, bf, i):
                sl = pl.ds(off, 16)
                x1 = buf[pl.ds(1, 1), sl] * _SCALE
                y1 = buf[pl.ds(2, 1), sl] * _SCALE
                x2 = buf[pl.ds(3, 1), sl] * _SCALE
                y2 = buf[pl.ds(4, 1), sl] * _SCALE
                osl = pl.ds(i * 16, 16)
                ob[pl.ds(0, 1), osl] = jnp.zeros((1, 16), jnp.float32) + bf
                ob[pl.ds(1, 1), osl] = jnp.clip(jnp.minimum(x1, x2), 0.0, 1.0)
                ob[pl.ds(2, 1), osl] = jnp.clip(jnp.minimum(y1, y2), 0.0, 1.0)
                ob[pl.ds(3, 1), osl] = jnp.clip(jnp.maximum(x1, x2), 0.0, 1.0)
                ob[pl.ds(4, 1), osl] = jnp.clip(jnp.maximum(y1, y2), 0.0, 1.0)

            @plsc.parallel_loop(0, i1, unroll=8)
            def _(i):
                vec_body(bin1.at[w], s0 + i * 16, bf0, i)

            @plsc.parallel_loop(i1, i2, unroll=1)
            def _(i):
                vec_body(bint.at[w], (i - i1) * 16, bf0, i)

            @plsc.parallel_loop(i2, _NV, unroll=4)
            def _(i):
                vec_body(bin2.at[w], (i - i2) * 16, bf1, i)

            if r + 2 < _NR:
                fetch(r + 2, w)
            out_copy(r, w).start()

    for r in range(max(_NR - 2, 0), _NR):
        _, act, *_ = params(r)
        pl.when(act)(out_copy(r, r % 2).wait)


def kernel(vit_output, input_images_or_features):
    del input_images_or_features
    vt = jnp.transpose(vit_output, (0, 2, 1))
    tail = lax.slice(vt, (0, 0, _QT), (_B, 8, _Q))
    out = _sc_body(vt, tail)
    return out.T


# speedup vs baseline: 2.9748x; 2.3549x over previous
import jax
import jax.numpy as jnp
from jax.experimental import pallas as pl
from jax.experimental.pallas import tpu as pltpu

_B = 16
_Q = 20000
_N = _B * _Q
_G = 4
_NG = _B // _G
_SCALE = 1.0 / 512.0


def _body(in_hbm, out_hbm, bin_, bout, semi, semo):
    def in_copy(b):
        return pltpu.make_async_copy(in_hbm.at[b], bin_.at[b % 2], semi.at[b % 2])

    def out_copy(g):
        return pltpu.make_async_copy(
            bout.at[g % 2], out_hbm.at[:, pl.ds(g * _G * _Q, _G * _Q)],
            semo.at[g % 2],
        )

    in_copy(0).start()
    in_copy(1).start()
    for b in range(_B):
        g, j = divmod(b, _G)
        if j == 0 and g >= 2:
            out_copy(g - 2).wait()
        in_copy(b).wait()
        v = bin_[b % 2]
        s = v * _SCALE
        mn = jnp.clip(jnp.minimum(s[1:3, :], s[3:5, :]), 0.0, 1.0)
        mx = jnp.clip(jnp.maximum(s[1:3, :], s[3:5, :]), 0.0, 1.0)
        brow = jnp.full((1, _Q), float(b), dtype=jnp.float32)
        res = jnp.concatenate([brow, mn, mx], axis=0)
        bw = bout.at[g % 2]
        bw[:, j * _Q:(j + 1) * _Q] = res
        if b + 2 < _B:
            in_copy(b + 2).start()
        if j == _G - 1:
            out_copy(g).start()
    out_copy(_NG - 2).wait()
    out_copy(_NG - 1).wait()


def kernel(vit_output, input_images_or_features):
    del input_images_or_features
    vt = jnp.transpose(vit_output, (0, 2, 1))
    out = pl.pallas_call(
        _body,
        in_specs=[pl.BlockSpec(memory_space=pl.ANY)],
        out_specs=pl.BlockSpec(memory_space=pl.ANY),
        out_shape=jax.ShapeDtypeStruct((5, _N), jnp.float32),
        scratch_shapes=[
            pltpu.VMEM((2, 8, _Q), jnp.float32),
            pltpu.VMEM((2, 5, _G * _Q), jnp.float32),
            pltpu.SemaphoreType.DMA((2,)),
            pltpu.SemaphoreType.DMA((2,)),
        ],
    )(vt)
    return out.T


# speedup vs baseline: 3.9060x; 1.3130x over previous
import jax
import jax.numpy as jnp
from jax.experimental import pallas as pl

_B = 16
_Q = 20000
_G = 4
_SCALE = 1.0 / 512.0


def _body(in_ref, out_ref):
    g = pl.program_id(0)
    for j in range(_G):
        v = in_ref[j]
        s = v * _SCALE
        mn = jnp.clip(jnp.minimum(s[1:3, :], s[3:5, :]), 0.0, 1.0)
        mx = jnp.clip(jnp.maximum(s[1:3, :], s[3:5, :]), 0.0, 1.0)
        bf = (g * _G + j).astype(jnp.float32)
        brow = jnp.zeros((1, _Q), jnp.float32) + bf
        res = jnp.concatenate([brow, mn, mx, mx, brow], axis=0)
        out_ref[:, j * _Q:(j + 1) * _Q] = res


def kernel(vit_output, input_images_or_features):
    del input_images_or_features
    vt = jnp.transpose(vit_output, (0, 2, 1))
    out = pl.pallas_call(
        _body,
        grid=(_B // _G,),
        in_specs=[pl.BlockSpec((_G, 8, _Q), lambda g: (g, 0, 0))],
        out_specs=pl.BlockSpec((8, _G * _Q), lambda g: (0, g)),
        out_shape=jax.ShapeDtypeStruct((8, _B * _Q), jnp.float32),
    )(vt)
    return out.T[:, :5]
